# baseline (speedup 1.0000x reference)
import jax
import jax.numpy as jnp
from jax.experimental import pallas as pl
from jax.experimental.pallas import tpu as pltpu

_N = 512
_H = 64
_B = 2
_P = 12
_Q = 12
_BN = _B * _N
_IP = 8


def _gts_kernel(nf, nfT, Wf, WfT, bf_r, bf_c, WpR, WpLT, bp_c, WoT, bo,
                xin, W0ru, b0ru, W0c, b0c, W1ru, b1ru, W1c, b1c,
                W2ru, b2ru, W2c, b2c, W3ru, b3ru, W3c, b3c,
                Wproj, bproj, out_ref):
    N, H, B, BN = _N, _H, _B, _BN
    bf16 = jnp.bfloat16

    feat = jnp.maximum(nfT[:] @ Wf[:] + bf_r[:], 0.0)
    featT = jnp.maximum(WfT[:] @ nf[:] + bf_c[:], 0.0)
    Rp = feat @ WpR[:]
    LpT = WpLT[:] @ featT + bp_c[:]
    wv = WoT[0:1, :] - WoT[1:2, :]
    c0 = bo[0, 0] - bo[0, 1]

    accT = jnp.zeros((N, N), jnp.float32)
    for h in range(H):
        accT = accT + wv[0, h] * jnp.maximum(
            Rp[:, h:h + 1] + LpT[h:h + 1, :], 0.0)
    AT = jax.nn.sigmoid(2.0 * (accT + c0))
    colsum = jnp.sum(AT, axis=0, keepdims=True)
    AhT = (AT / (colsum + 1e-8)).astype(bf16)

    def hopall(X):
        return jnp.concatenate(
            [jnp.dot(X[:, b * N:(b + 1) * N], AhT,
                     preferred_element_type=jnp.float32).astype(bf16)
             for b in range(B)], axis=1)

    def wmm(WT, p0, p1, p2, bias_c):
        W = WT[:]
        outs = []
        for b in range(B):
            s = slice(b * N, (b + 1) * N)
            c3 = jnp.concatenate([p0[:, s], p1[:, s], p2[:, s]], axis=0)
            outs.append(jnp.dot(W, c3, preferred_element_type=jnp.float32))
        return jnp.concatenate(outs, axis=1) + bias_c[:]

    def cell(xt, h, WruT, bruT, WcT, bcT, ind):
        hb = h.astype(bf16)
        cat1 = jnp.concatenate([xt, hb], axis=0)
        x1 = hopall(cat1)
        x2 = hopall(x1)
        ru = jax.nn.sigmoid(wmm(WruT, cat1, x1, x2, bruT))
        r = ru[0:H, :]
        u = ru[H:2 * H, :]
        s = (r * h).astype(bf16)
        cat2 = jnp.concatenate([xt, s], axis=0)
        s1 = hopall(s)
        x1b = jnp.concatenate([x1[0:ind, :], s1], axis=0)
        s2 = hopall(s1)
        x2b = jnp.concatenate([x2[0:ind, :], s2], axis=0)
        c = jnp.tanh(wmm(WcT, cat2, x1b, x2b, bcT))
        return u * h + (1.0 - u) * c

    z = jnp.zeros((H, BN), jnp.float32)
    h0, h1 = z, z
    for t in range(_P):
        xt = xin[t * _IP:(t + 1) * _IP, :].astype(bf16)
        h0 = cell(xt, h0, W0ru, b0ru, W0c, b0c, _IP)
        h1 = cell(h0.astype(bf16), h1, W1ru, b1ru, W1c, b1c, H)

    g0, g1 = h0, h1
    zq = jnp.zeros((_IP - 1, BN), bf16)
    xq = jnp.zeros((1, BN), bf16)
    for q in range(_Q):
        xq8 = jnp.concatenate([xq, zq], axis=0)
        g0 = cell(xq8, g0, W2ru, b2ru, W2c, b2c, _IP)
        g1 = cell(g0.astype(bf16), g1, W3ru, b3ru, W3c, b3c, H)
        pr = jnp.sum(g1 * Wproj[:], axis=0, keepdims=True) + bproj[0, 0]
        xq = pr.astype(bf16)
        for b in range(B):
            out_ref[0, q * B + b, :] = pr[0, b * N:(b + 1) * N]


def _pad_narrow(W):
    F = 1 + _H
    blocks = []
    for k in range(3):
        blk = W[k * F:(k + 1) * F]
        blocks.append(jnp.concatenate(
            [blk[0:1], jnp.zeros((_IP - 1, W.shape[1]), W.dtype), blk[1:]],
            axis=0))
    return jnp.concatenate(blocks, axis=0)


@jax.jit
def _run(x, node_features, Wf, bf, Wp, bp, Wo, bo,
         enc0_Wru, enc0_bru, enc0_Wc, enc0_bc,
         enc1_Wru, enc1_bru, enc1_Wc, enc1_bc,
         dec0_Wru, dec0_bru, dec0_Wc, dec0_bc,
         dec1_Wru, dec1_bru, dec1_Wc, dec1_bc,
         Wproj, bproj):
    bf16 = jnp.bfloat16
    col = lambda v: v.reshape(-1, 1)
    tb = lambda W: W.T.astype(bf16)
    xfm = jnp.transpose(x[..., 0], (1, 0, 2)).reshape(_P, 1, 4 * _N)
    xin = jnp.concatenate(
        [xfm, jnp.zeros((_P, _IP - 1, 4 * _N), jnp.float32)], axis=1
    ).reshape(_P * _IP, 4 * _N)

    args = (node_features, node_features.T, Wf, Wf.T, bf.reshape(1, -1),
            col(bf), Wp[_H:2 * _H, :], Wp[0:_H, :].T, col(bp), Wo.T,
            bo.reshape(1, -1), xin,
            tb(_pad_narrow(enc0_Wru)), col(enc0_bru),
            tb(_pad_narrow(enc0_Wc)), col(enc0_bc),
            tb(enc1_Wru), col(enc1_bru), tb(enc1_Wc), col(enc1_bc),
            tb(_pad_narrow(dec0_Wru)), col(dec0_bru),
            tb(_pad_narrow(dec0_Wc)), col(dec0_bc),
            tb(dec1_Wru), col(dec1_bru), tb(dec1_Wc), col(dec1_bc),
            Wproj, bproj.reshape(1, 1))

    full = lambda a: pl.BlockSpec(a.shape, lambda i: (0,) * a.ndim)
    in_specs = [full(a) for a in args]
    in_specs[11] = pl.BlockSpec((_P * _IP, _BN), lambda i: (0, i))

    out = pl.pallas_call(
        _gts_kernel,
        grid=(2,),
        in_specs=in_specs,
        out_specs=pl.BlockSpec((1, _Q * _B, _N), lambda i: (i, 0, 0)),
        out_shape=jax.ShapeDtypeStruct((2, _Q * _B, _N), jnp.float32),
        compiler_params=pltpu.CompilerParams(
            dimension_semantics=("parallel",)),
    )(*args)
    return jnp.transpose(out.reshape(2, _Q, _B, _N), (0, 2, 1, 3)
                         ).reshape(4, _Q, _N)


def kernel(x, As, ycl, iteration, node_features, Wf, bf, Wp, bp, Wo, bo,
           enc0_Wru, enc0_bru, enc0_Wc, enc0_bc,
           enc1_Wru, enc1_bru, enc1_Wc, enc1_bc,
           dec0_Wru, dec0_bru, dec0_Wc, dec0_bc,
           dec1_Wru, dec1_bru, dec1_Wc, dec1_bc,
           Wproj, bproj):
    return _run(x, node_features, Wf, bf, Wp, bp, Wo, bo,
                enc0_Wru, enc0_bru, enc0_Wc, enc0_bc,
                enc1_Wru, enc1_bru, enc1_Wc, enc1_bc,
                dec0_Wru, dec0_bru, dec0_Wc, dec0_bc,
                dec1_Wru, dec1_bru, dec1_Wc, dec1_bc,
                Wproj, bproj)


# speedup vs baseline: 1.6163x; 1.6163x over previous
import jax
import jax.numpy as jnp
from jax.experimental import pallas as pl

_N = 512
_H = 64
_B = 4
_P = 12
_Q = 12
_BN = _B * _N
_IP = 8


def _gts_kernel(nf, nfT, Wf, WfT, bf_r, bf_c, WpR, WpLT, bp_c, WoT, bo,
                xin, W0ru, b0ru, W0c, b0c, W1ru, b1ru, W1c, b1c,
                W2ru, b2ru, W2c, b2c, W3ru, b3ru, W3c, b3c,
                Wproj, bproj, out_ref):
    N, H, B, BN = _N, _H, _B, _BN
    bf16 = jnp.bfloat16

    feat = jnp.maximum(nfT[:] @ Wf[:] + bf_r[:], 0.0)
    featT = jnp.maximum(WfT[:] @ nf[:] + bf_c[:], 0.0)
    Rp = feat @ WpR[:]
    LpT = WpLT[:] @ featT + bp_c[:]
    wv = WoT[0:1, :] - WoT[1:2, :]
    c0 = bo[0, 0] - bo[0, 1]

    accT = jnp.zeros((N, N), jnp.float32)
    for h in range(H):
        accT = accT + wv[0, h] * jnp.maximum(
            Rp[:, h:h + 1] + LpT[h:h + 1, :], 0.0)
    AT = jax.nn.sigmoid(2.0 * (accT + c0))
    colsum = jnp.sum(AT, axis=0, keepdims=True)
    AhT = (AT / (colsum + 1e-8)).astype(bf16)

    def hopall(X):
        return jnp.concatenate(
            [jnp.dot(X[:, b * N:(b + 1) * N], AhT,
                     preferred_element_type=jnp.float32).astype(bf16)
             for b in range(B)], axis=1)

    def wmm(WT, p0, p1, p2, bias_c):
        W = WT[:]
        outs = []
        for b in range(B):
            s = slice(b * N, (b + 1) * N)
            c3 = jnp.concatenate([p0[:, s], p1[:, s], p2[:, s]], axis=0)
            outs.append(jnp.dot(W, c3, preferred_element_type=jnp.float32))
        return jnp.concatenate(outs, axis=1) + bias_c[:]

    def cell(xt, h, WruT, bruT, WcT, bcT, ind):
        hb = h.astype(bf16)
        cat1 = jnp.concatenate([xt, hb], axis=0)
        x1 = hopall(cat1)
        x2 = hopall(x1)
        ru = jax.nn.sigmoid(wmm(WruT, cat1, x1, x2, bruT))
        r = ru[0:H, :]
        u = ru[H:2 * H, :]
        s = (r * h).astype(bf16)
        cat2 = jnp.concatenate([xt, s], axis=0)
        s1 = hopall(s)
        x1b = jnp.concatenate([x1[0:ind, :], s1], axis=0)
        s2 = hopall(s1)
        x2b = jnp.concatenate([x2[0:ind, :], s2], axis=0)
        c = jnp.tanh(wmm(WcT, cat2, x1b, x2b, bcT))
        return u * h + (1.0 - u) * c

    z = jnp.zeros((H, BN), jnp.float32)
    h0, h1 = z, z
    for t in range(_P):
        xt = xin[t * _IP:(t + 1) * _IP, :].astype(bf16)
        h0 = cell(xt, h0, W0ru, b0ru, W0c, b0c, _IP)
        h1 = cell(h0.astype(bf16), h1, W1ru, b1ru, W1c, b1c, H)

    g0, g1 = h0, h1
    zq = jnp.zeros((_IP - 1, BN), bf16)
    xq = jnp.zeros((1, BN), bf16)
    for q in range(_Q):
        xq8 = jnp.concatenate([xq, zq], axis=0)
        g0 = cell(xq8, g0, W2ru, b2ru, W2c, b2c, _IP)
        g1 = cell(g0.astype(bf16), g1, W3ru, b3ru, W3c, b3c, H)
        pr = jnp.sum(g1 * Wproj[:], axis=0, keepdims=True) + bproj[0, 0]
        xq = pr.astype(bf16)
        for b in range(B):
            out_ref[q * B + b:q * B + b + 1, :] = pr[:, b * N:(b + 1) * N]


def _pad_narrow(W):
    F = 1 + _H
    blocks = []
    for k in range(3):
        blk = W[k * F:(k + 1) * F]
        blocks.append(jnp.concatenate(
            [blk[0:1], jnp.zeros((_IP - 1, W.shape[1]), W.dtype), blk[1:]],
            axis=0))
    return jnp.concatenate(blocks, axis=0)


@jax.jit
def _run(x, node_features, Wf, bf, Wp, bp, Wo, bo,
         enc0_Wru, enc0_bru, enc0_Wc, enc0_bc,
         enc1_Wru, enc1_bru, enc1_Wc, enc1_bc,
         dec0_Wru, dec0_bru, dec0_Wc, dec0_bc,
         dec1_Wru, dec1_bru, dec1_Wc, dec1_bc,
         Wproj, bproj):
    bf16 = jnp.bfloat16
    col = lambda v: v.reshape(-1, 1)
    tb = lambda W: W.T.astype(bf16)
    xfm = jnp.transpose(x[..., 0], (1, 0, 2)).reshape(_P, 1, _BN)
    xin = jnp.concatenate(
        [xfm, jnp.zeros((_P, _IP - 1, _BN), jnp.float32)], axis=1
    ).reshape(_P * _IP, _BN)

    out = pl.pallas_call(
        _gts_kernel,
        out_shape=jax.ShapeDtypeStruct((_Q * _B, _N), jnp.float32),
    )(node_features, node_features.T, Wf, Wf.T, bf.reshape(1, -1), col(bf),
      Wp[_H:2 * _H, :], Wp[0:_H, :].T, col(bp), Wo.T, bo.reshape(1, -1),
      xin,
      tb(_pad_narrow(enc0_Wru)), col(enc0_bru),
      tb(_pad_narrow(enc0_Wc)), col(enc0_bc),
      tb(enc1_Wru), col(enc1_bru), tb(enc1_Wc), col(enc1_bc),
      tb(_pad_narrow(dec0_Wru)), col(dec0_bru),
      tb(_pad_narrow(dec0_Wc)), col(dec0_bc),
      tb(dec1_Wru), col(dec1_bru), tb(dec1_Wc), col(dec1_bc),
      Wproj, bproj.reshape(1, 1))
    return jnp.transpose(out.reshape(_Q, _B, _N), (1, 0, 2))


def kernel(x, As, ycl, iteration, node_features, Wf, bf, Wp, bp, Wo, bo,
           enc0_Wru, enc0_bru, enc0_Wc, enc0_bc,
           enc1_Wru, enc1_bru, enc1_Wc, enc1_bc,
           dec0_Wru, dec0_bru, dec0_Wc, dec0_bc,
           dec1_Wru, dec1_bru, dec1_Wc, dec1_bc,
           Wproj, bproj):
    return _run(x, node_features, Wf, bf, Wp, bp, Wo, bo,
                enc0_Wru, enc0_bru, enc0_Wc, enc0_bc,
                enc1_Wru, enc1_bru, enc1_Wc, enc1_bc,
                dec0_Wru, dec0_bru, dec0_Wc, dec0_bc,
                dec1_Wru, dec1_bru, dec1_Wc, dec1_bc,
                Wproj, bproj)
